# baseline (speedup 1.0000x reference)
import functools

import jax
import jax.numpy as jnp
from jax import lax
from jax.experimental import pallas as pl
from jax.experimental.pallas import tpu as pltpu
from jax.experimental.pallas import tpu_sc as plsc


def _transpose_table(wt):
    D, V = wt.shape
    BLK = 8192
    grid = (V + BLK - 1) // BLK

    def body(wt_ref, out_ref):
        out_ref[...] = wt_ref[...].T

    return pl.pallas_call(
        body,
        grid=(grid,),
        in_specs=[pl.BlockSpec((D, BLK), lambda i: (0, i))],
        out_specs=pl.BlockSpec((BLK, D), lambda i: (i, 0)),
        out_shape=jax.ShapeDtypeStruct((V, D), jnp.float32),
    )(wt)


def kernel(input_tensor, weight):
    B0, B1 = input_tensor.shape
    V, D = weight.shape
    B = B0 * B1

    info = plsc.get_sparse_core_info()
    NC, NS = info.num_cores, info.num_subcores
    NW = NC * NS

    KW = 128
    n_blocks = B // KW
    blocks_w = n_blocks // NW
    bblks = B0 // KW
    NBUF = 3

    w_t = weight.T
    idx2 = input_tensor.T.reshape(n_blocks, KW).astype(jnp.int32)

    table = _transpose_table(w_t)

    mesh = plsc.VectorSubcoreMesh(core_axis_name="c", subcore_axis_name="s")

    @functools.partial(
        pl.kernel,
        mesh=mesh,
        out_type=jax.ShapeDtypeStruct((B1, D, B0), jnp.float32),
        scratch_types=[
            pltpu.VMEM((blocks_w, KW), jnp.int32),
            pltpu.VMEM((NBUF, KW, D), jnp.float32),
            pltpu.VMEM((NBUF, D, KW), jnp.float32),
            pltpu.SemaphoreType.DMA,
            pltpu.SemaphoreType.DMA,
            pltpu.SemaphoreType.DMA,
            pltpu.SemaphoreType.DMA,
            pltpu.SemaphoreType.DMA,
            pltpu.SemaphoreType.DMA,
        ],
        compiler_params=pltpu.CompilerParams(
            use_tc_tiling_on_sc=False, needs_layout_passes=False),
    )
    def emb(idx_hbm, table_hbm, out_hbm, idx_v, rows_v, tr_v,
            g0, g1, g2, o0, o1, o2):
        gsem = (g0, g1, g2)
        osem = (o0, o1, o2)
        wid = lax.axis_index("s") * NC + lax.axis_index("c")
        blk0 = wid * blocks_w

        pltpu.sync_copy(idx_hbm.at[pl.ds(blk0, blocks_w), :], idx_v)

        def fire_gather(i, r):
            pltpu.make_async_copy(
                table_hbm.at[idx_v.at[i]], rows_v.at[r], gsem[r]
            ).start()

        def drain_gather(r):
            pltpu.make_async_copy(
                table_hbm.at[pl.ds(0, KW), :], rows_v.at[r], gsem[r]
            ).wait()

        def transpose_block(r):
            src = rows_v.at[r]
            dst = tr_v.at[r]

            def dbody(d, carry):
                col = jnp.full((16,), d, jnp.int32)
                for j0 in range(0, KW, 16):
                    row = j0 + lax.iota(jnp.int32, 16)
                    vals = plsc.load_gather(src, [row, col])
                    dst[d, pl.ds(j0, 16)] = vals
                return carry

            lax.fori_loop(0, D, dbody, 0)

        def fire_write(i, r):
            gi = blk0 + i
            t = gi // bblks
            b0 = (gi % bblks) * KW
            pltpu.make_async_copy(
                tr_v.at[r], out_hbm.at[t, :, pl.ds(b0, KW)], osem[r]
            ).start()

        def drain_write(r):
            pltpu.make_async_copy(
                tr_v.at[r], out_hbm.at[0, :, pl.ds(0, KW)], osem[r]
            ).wait()

        def steady(i, r):
            drain_write(r)
            fire_gather(i, r)
            p = (r + NBUF - 1) % NBUF
            drain_gather(p)
            transpose_block(p)
            fire_write(i - 1, p)

        fire_gather(0, 0)
        fire_gather(1, 1)
        drain_gather(0)
        transpose_block(0)
        fire_write(0, 0)
        fire_gather(2, 2)
        drain_gather(1)
        transpose_block(1)
        fire_write(1, 1)

        n_groups = (blocks_w - NBUF) // NBUF

        def g3(g, carry):
            base = NBUF + g * NBUF
            steady(base + 0, 0)
            steady(base + 1, 1)
            steady(base + 2, 2)
            return carry

        lax.fori_loop(0, n_groups, g3, 0)

        for i in range(NBUF + n_groups * NBUF, blocks_w):
            steady(i, i % NBUF)

        last = blocks_w - 1
        p = last % NBUF
        drain_gather(p)
        transpose_block(p)
        fire_write(last, p)
        drain_write(0)
        drain_write(1)
        drain_write(2)

    out3 = emb(idx2, table)
    return jnp.transpose(out3, (2, 0, 1))


# speedup vs baseline: 2.9811x; 2.9811x over previous
import functools

import jax
import jax.numpy as jnp
from jax import lax
from jax.experimental import pallas as pl
from jax.experimental.pallas import tpu as pltpu
from jax.experimental.pallas import tpu_sc as plsc


def _transpose_table(wt):
    D, V = wt.shape
    BLK = 8192
    grid = (V + BLK - 1) // BLK

    def body(wt_ref, out_ref):
        out_ref[:, 0:D] = wt_ref[...].T

    return pl.pallas_call(
        body,
        grid=(grid,),
        in_specs=[pl.BlockSpec((D, BLK), lambda i: (0, i))],
        out_specs=pl.BlockSpec((BLK, 2 * D), lambda i: (i, 0)),
        out_shape=jax.ShapeDtypeStruct((V, 2 * D), jnp.float32),
    )(wt)


def _transpose_out(paired, B1, B0, D):
    def body(in_ref, out_ref):
        out_ref[0] = in_ref[:, 0:D].T

    return pl.pallas_call(
        body,
        grid=(B1,),
        in_specs=[pl.BlockSpec((B0, 2 * D), lambda i: (i, 0))],
        out_specs=pl.BlockSpec((1, D, B0), lambda i: (i, 0, 0)),
        out_shape=jax.ShapeDtypeStruct((B1, D, B0), jnp.float32),
    )(paired)


def _sc_gather(idx2, table, B, D):
    info = plsc.get_sparse_core_info()
    NC, NS = info.num_cores, info.num_subcores
    NW = NC * NS

    KW = 128
    CH = 4
    ROWS = CH * KW
    assert B % (NW * 2 * ROWS) == 0
    per_w = B // NW
    n_chunks = per_w // ROWS
    H = n_chunks // 2
    idx_rows_w = per_w // KW

    mesh = plsc.VectorSubcoreMesh(core_axis_name="c", subcore_axis_name="s")

    @functools.partial(
        pl.kernel,
        mesh=mesh,
        out_type=jax.ShapeDtypeStruct((B, 2 * D), jnp.float32),
        scratch_types=[
            pltpu.VMEM((idx_rows_w, KW), jnp.int32),
            pltpu.VMEM((ROWS, D), jnp.float32),
            pltpu.VMEM((ROWS, D), jnp.float32),
            pltpu.SemaphoreType.DMA,
            pltpu.SemaphoreType.DMA,
            pltpu.SemaphoreType.DMA,
            pltpu.SemaphoreType.DMA,
        ],
        compiler_params=pltpu.CompilerParams(use_tc_tiling_on_sc=False),
    )
    def emb(idx_hbm, table_hbm, out_hbm, idx_v, rows0, rows1,
            g0, g1, o0, o1):
        wid = lax.axis_index("s") * NC + lax.axis_index("c")
        out_row0 = wid * per_w

        pltpu.sync_copy(idx_hbm.at[pl.ds(wid * idx_rows_w, idx_rows_w)],
                        idx_v)

        def fire_gathers(chunk, rows_v, sem):
            for i in range(CH):
                pltpu.make_async_copy(
                    table_hbm.at[idx_v.at[chunk * CH + i]],
                    rows_v.at[pl.ds(i * KW, KW), :],
                    sem,
                ).start()

        def drain_gathers(rows_v, sem):
            pltpu.make_async_copy(
                table_hbm.at[pl.ds(0, ROWS), :], rows_v, sem).wait()

        def fire_write(chunk, rows_v, sem):
            pltpu.make_async_copy(
                rows_v,
                out_hbm.at[pl.ds(out_row0 + chunk * ROWS, ROWS),
                           pl.ds(0, D)],
                sem,
            ).start()

        def drain_write(rows_v, sem):
            pltpu.make_async_copy(
                rows_v,
                out_hbm.at[pl.ds(0, ROWS), pl.ds(0, D)], sem).wait()

        fire_gathers(0, rows0, g0)
        fire_gathers(1, rows1, g1)

        def body(j, carry):
            drain_gathers(rows0, g0)
            fire_write(2 * j - 2, rows0, o0)
            drain_write(rows0, o0)
            fire_gathers(2 * j, rows0, g0)
            drain_gathers(rows1, g1)
            fire_write(2 * j - 1, rows1, o1)
            drain_write(rows1, o1)
            fire_gathers(2 * j + 1, rows1, g1)
            return carry

        lax.fori_loop(1, H, body, 0)

        drain_gathers(rows0, g0)
        fire_write(2 * H - 2, rows0, o0)
        drain_gathers(rows1, g1)
        fire_write(2 * H - 1, rows1, o1)
        drain_write(rows0, o0)
        drain_write(rows1, o1)

    return emb(idx2, table)


def kernel(input_tensor, weight):
    B0, B1 = input_tensor.shape
    V, D = weight.shape
    B = B0 * B1
    KW = 128

    idx2 = (2 * input_tensor.T.astype(jnp.int32)).reshape(B // KW, KW)

    w_t = weight.T
    table = _transpose_table(w_t)
    table2 = table.reshape(2 * V, D)
    paired = _sc_gather(idx2, table2, B, D)
    out3 = _transpose_out(paired, B1, B0, D)

    return jnp.transpose(out3, (2, 0, 1))


# speedup vs baseline: 3.0309x; 1.0167x over previous
import functools

import jax
import jax.numpy as jnp
from jax import lax
from jax.experimental import pallas as pl
from jax.experimental.pallas import tpu as pltpu
from jax.experimental.pallas import tpu_sc as plsc


def _transpose_table(wt):
    D, V = wt.shape
    BLK = 8192
    grid = (V + BLK - 1) // BLK

    def body(wt_ref, out_ref):
        out_ref[:, 0:D] = wt_ref[...].T

    return pl.pallas_call(
        body,
        grid=(grid,),
        in_specs=[pl.BlockSpec((D, BLK), lambda i: (0, i))],
        out_specs=pl.BlockSpec((BLK, 2 * D), lambda i: (i, 0)),
        out_shape=jax.ShapeDtypeStruct((V, 2 * D), jnp.float32),
    )(wt)


def _transpose_out(rows, half, prev, T1, B1, B0, D):
    def body(*refs):
        in_ref, out_ref = refs[0], refs[-1]
        out_ref[0] = in_ref[:, 0:D].T

    in_specs = [pl.BlockSpec((B0, 2 * D), lambda i: (i, 0))]
    args = (rows,)
    aliases = {}
    if prev is not None:
        in_specs.append(pl.BlockSpec(memory_space=pl.ANY))
        args = (rows, prev)
        aliases = {1: 0}
    return pl.pallas_call(
        body,
        grid=(T1,),
        in_specs=in_specs,
        out_specs=pl.BlockSpec((1, D, B0), lambda i: (i + half * T1, 0, 0)),
        out_shape=jax.ShapeDtypeStruct((B1, D, B0), jnp.float32),
        input_output_aliases=aliases,
    )(*args)


def _sc_gather(idx2, table, B, D):
    info = plsc.get_sparse_core_info()
    NC, NS = info.num_cores, info.num_subcores
    NW = NC * NS

    KW = 128
    CH = 5
    ROWS = CH * KW
    per_w = B // NW
    n_chunks = per_w // ROWS
    assert per_w % ROWS == 0 and n_chunks % 2 == 0
    H = n_chunks // 2
    idx_rows_w = per_w // KW

    mesh = plsc.VectorSubcoreMesh(core_axis_name="c", subcore_axis_name="s")

    @functools.partial(
        pl.kernel,
        mesh=mesh,
        out_type=jax.ShapeDtypeStruct((B, 2 * D), jnp.float32),
        scratch_types=[
            pltpu.VMEM((idx_rows_w, KW), jnp.int32),
            pltpu.VMEM((ROWS, D), jnp.float32),
            pltpu.VMEM((ROWS, D), jnp.float32),
            pltpu.SemaphoreType.DMA,
            pltpu.SemaphoreType.DMA,
            pltpu.SemaphoreType.DMA,
            pltpu.SemaphoreType.DMA,
        ],
        compiler_params=pltpu.CompilerParams(use_tc_tiling_on_sc=False),
    )
    def emb(idx_hbm, table_hbm, out_hbm, idx_v, rows0, rows1,
            g0, g1, o0, o1):
        wid = lax.axis_index("s") * NC + lax.axis_index("c")
        out_row0 = wid * per_w

        pltpu.sync_copy(idx_hbm.at[pl.ds(wid * idx_rows_w, idx_rows_w)],
                        idx_v)

        def fire_gathers(chunk, rows_v, sem):
            for i in range(CH):
                pltpu.make_async_copy(
                    table_hbm.at[idx_v.at[chunk * CH + i]],
                    rows_v.at[pl.ds(i * KW, KW), :],
                    sem,
                ).start()

        def drain_gathers(rows_v, sem):
            pltpu.make_async_copy(
                table_hbm.at[pl.ds(0, ROWS), :], rows_v, sem).wait()

        def fire_write(chunk, rows_v, sem):
            pltpu.make_async_copy(
                rows_v,
                out_hbm.at[pl.ds(out_row0 + chunk * ROWS, ROWS),
                           pl.ds(0, D)],
                sem,
            ).start()

        def drain_write(rows_v, sem):
            pltpu.make_async_copy(
                rows_v,
                out_hbm.at[pl.ds(0, ROWS), pl.ds(0, D)], sem).wait()

        fire_gathers(0, rows0, g0)
        fire_gathers(1, rows1, g1)

        def body(j, carry):
            drain_gathers(rows0, g0)
            fire_write(2 * j - 2, rows0, o0)
            drain_write(rows0, o0)
            fire_gathers(2 * j, rows0, g0)
            drain_gathers(rows1, g1)
            fire_write(2 * j - 1, rows1, o1)
            drain_write(rows1, o1)
            fire_gathers(2 * j + 1, rows1, g1)
            return carry

        lax.fori_loop(1, H, body, 0)

        drain_gathers(rows0, g0)
        fire_write(2 * H - 2, rows0, o0)
        drain_gathers(rows1, g1)
        fire_write(2 * H - 1, rows1, o1)
        drain_write(rows0, o0)
        drain_write(rows1, o1)

    return emb(idx2, table)


def kernel(input_tensor, weight):
    B0, B1 = input_tensor.shape
    V, D = weight.shape
    B = B0 * B1
    KW = 128
    T1 = B1 // 2

    idx2 = (2 * input_tensor.T.astype(jnp.int32)).reshape(B // KW, KW)
    half_rows = B // (2 * KW)

    w_t = weight.T
    table = _transpose_table(w_t)
    table2 = table.reshape(2 * V, D)

    g1 = _sc_gather(idx2[:half_rows], table2, B // 2, D)
    g2 = _sc_gather(idx2[half_rows:], table2, B // 2, D)
    o1 = _transpose_out(g1, 0, None, T1, B1, B0, D)
    o2 = _transpose_out(g2, 1, o1, T1, B1, B0, D)

    return jnp.transpose(o2, (2, 0, 1))
